# baseline (speedup 1.0000x reference)
import functools

import jax
import jax.numpy as jnp
from jax.experimental import pallas as pl
from jax.experimental.pallas import tpu as pltpu

_NU = 25000
_NI = 25000
_N = _NU + _NI
_E = 800000
_BLK = 1000


def _final_add_body(a_ref, b_ref, c_ref, d_ref, o_ref):
    o_ref[...] = a_ref[...] + b_ref[...] + c_ref[...] + d_ref[...]


def _final_add(a, b, c, d):
    n, k = a.shape
    grid = n // _BLK
    spec = pl.BlockSpec((_BLK, k), lambda i: (i, 0))
    return pl.pallas_call(
        _final_add_body,
        grid=(grid,),
        in_specs=[spec, spec, spec, spec],
        out_specs=spec,
        out_shape=jax.ShapeDtypeStruct((n, k), a.dtype),
    )(a, b, c, d)


def kernel(edge_index, Gu, Gi, feat_visual, Gum_visual, proj_W_visual, proj_b_visual, prop_W_visual_0, lin_W_visual_0, lin_b_visual_0, g_W_visual_0, g_b_visual_0, prop_W_visual_1, lin_W_visual_1, lin_b_visual_1, g_W_visual_1, g_b_visual_1, feat_textual, Gum_textual, proj_W_textual, proj_b_textual, prop_W_textual_0, lin_W_textual_0, lin_b_textual_0, g_W_textual_0, g_b_textual_0, prop_W_textual_1, lin_W_textual_1, lin_b_textual_1, g_W_textual_1, g_b_textual_1):
    kw = dict(locals())
    src, dst = edge_index[0], edge_index[1]
    ego = jnp.concatenate([Gu, Gi], axis=0)
    outs = []
    for m in ("visual", "textual"):
        proj = kw[f"feat_{m}"] @ kw[f"proj_W_{m}"].T + kw[f"proj_b_{m}"]
        x = jnp.concatenate([kw[f"Gum_{m}"], proj], axis=0)
        nrm = jnp.linalg.norm(x, axis=1, keepdims=True)
        x = x / jnp.maximum(nrm, 1e-12)
        for l in range(2):
            y = x @ kw[f"prop_W_{m}_{l}"]
            h = jax.ops.segment_sum(y[src], dst, num_segments=_N)
            x_hat = x @ kw[f"lin_W_{m}_{l}"].T + kw[f"lin_b_{m}_{l}"] + ego
            x = h @ kw[f"g_W_{m}_{l}"].T + kw[f"g_b_{m}_{l}"] + x_hat
        outs.append(x)
    x_all = _final_add(outs[0], outs[1], jnp.zeros_like(outs[0]), jnp.zeros_like(outs[0]))
    return x_all[:_NU], x_all[_NU:]


# speedup vs baseline: 2.8443x; 2.8443x over previous
import functools

import jax
import jax.numpy as jnp
from jax import lax
from jax.experimental import pallas as pl
from jax.experimental.pallas import tpu as pltpu
from jax.experimental.pallas import tpu_sc as plsc

_NU = 25000
_NI = 25000
_N = _NU + _NI
_E = 800000
_K = 64
_BLK = 1000

_HALF = 25000
_ACC = 25088
_ROWS_PER_TILE = _ACC // 16
_EROW = 128
_GRP = 2
_EPAD = 819200
_NGRP = _EPAD // (_EROW * _GRP)
_WB = 1000


def _seg_body(z, srcm, dstm, zz, out, sidx, dloc, rows, acc, sem):
    c = lax.axis_index("c")
    s = lax.axis_index("s")
    base = c * _HALF

    pltpu.sync_copy(zz.at[pl.ds(s * _ROWS_PER_TILE, _ROWS_PER_TILE)],
                    acc.at[pl.ds(s * _ROWS_PER_TILE, _ROWS_PER_TILE)])
    plsc.subcore_barrier()

    iota = lax.iota(jnp.int32, 16)

    def chunk(k, carry):
        g = s + 16 * k
        r0 = _GRP * g
        pltpu.sync_copy(srcm.at[pl.ds(r0, _GRP)], sidx)
        pltpu.sync_copy(dstm.at[pl.ds(r0, _GRP)], dloc)
        for r in range(_GRP):
            for i in range(_EROW // 16):
                d = dloc[r, pl.ds(i * 16, 16)]
                loc = d - base
                m = (loc >= 0) & (loc < _HALF)
                dloc[r, pl.ds(i * 16, 16)] = jnp.where(m, loc, _HALF + iota)
        cps = []
        for r in range(_GRP):
            cps.append(pltpu.async_copy(
                z.at[sidx.at[r]], rows.at[pl.ds(r * _EROW, _EROW)], sem))
        for cp in cps:
            cp.wait()
        for r in range(_GRP):
            pltpu.sync_copy(rows.at[pl.ds(r * _EROW, _EROW)],
                            acc.at[dloc.at[r]], add=True)
        return carry

    lax.fori_loop(0, _NGRP // 16, chunk, 0)
    plsc.subcore_barrier()

    for k2 in range(( _HALF // _WB + 15) // 16):
        g = s + 16 * k2

        @pl.when(g < _HALF // _WB)
        def _():
            pltpu.sync_copy(acc.at[pl.ds(g * _WB, _WB)],
                            out.at[pl.ds(base + g * _WB, _WB)])


@jax.jit
def _segment_sum_sc(z, src2d, dst2d, zz):
    mesh = plsc.VectorSubcoreMesh(core_axis_name="c", subcore_axis_name="s")
    return pl.kernel(
        _seg_body,
        out_type=jax.ShapeDtypeStruct((_N, _K), jnp.float32),
        mesh=mesh,
        scratch_types=[
            pltpu.VMEM((_GRP, _EROW), jnp.int32),
            pltpu.VMEM((_GRP, _EROW), jnp.int32),
            pltpu.VMEM((_GRP * _EROW, _K), jnp.float32),
            pltpu.VMEM_SHARED((_ACC, _K), jnp.float32),
            pltpu.SemaphoreType.DMA,
        ],
        compiler_params=pltpu.CompilerParams(use_tc_tiling_on_sc=False),
    )(z, src2d, dst2d, zz)


def _dotT(a, b):
    return lax.dot_general(a, b, (((1,), (1,)), ((), ())),
                           preferred_element_type=jnp.float32)


def _normalize(x):
    nrm = jnp.sqrt(jnp.sum(x * x, axis=1, keepdims=True))
    return x / jnp.maximum(nrm, 1e-12)


def _prep_items_body(fv, wv, bv, ft, wt, bt, ov, ot):
    ov[...] = _normalize(_dotT(fv[...], wv[...]) + bv[...])
    ot[...] = _normalize(_dotT(ft[...], wt[...]) + bt[...])


def _prep_users_body(gv, gt, ov, ot):
    ov[...] = _normalize(gv[...])
    ot[...] = _normalize(gt[...])


def _layer_core(x, pw, gw, lw, bias, ego):
    weff = _dotT(pw, gw)
    z = jnp.dot(x, weff, preferred_element_type=jnp.float32)
    xh = _dotT(x, lw) + bias + ego
    return z, xh


def _layer0_body(xv, pwv, gwv, lwv, bv, xt, pwt, gwt, lwt, bt, ego,
                 zv, xhv, zt, xht):
    zv[...], xhv[...] = _layer_core(xv[...], pwv[...], gwv[...], lwv[...],
                                    bv[...], ego[...])
    zt[...], xht[...] = _layer_core(xt[...], pwt[...], gwt[...], lwt[...],
                                    bt[...], ego[...])


def _layer1_body(sv, xpv, pwv, gwv, lwv, bv, st, xpt, pwt, gwt, lwt, bt, ego,
                 zv, xhv, zt, xht):
    zv[...], xhv[...] = _layer_core(sv[...] + xpv[...], pwv[...], gwv[...],
                                    lwv[...], bv[...], ego[...])
    zt[...], xht[...] = _layer_core(st[...] + xpt[...], pwt[...], gwt[...],
                                    lwt[...], bt[...], ego[...])


def _final_body(a, b, c, d, o):
    o[...] = a[...] + b[...] + c[...] + d[...]


def _row_spec(blk, k):
    return pl.BlockSpec((blk, k), lambda i: (i, 0))


def _full_spec(r, k):
    return pl.BlockSpec((r, k), lambda i: (0, 0))


def _prep_items(fv, wv, bv, ft, wt, bt):
    grid = _NI // _BLK
    out = jax.ShapeDtypeStruct((_NI, _K), jnp.float32)
    return pl.pallas_call(
        _prep_items_body,
        grid=(grid,),
        in_specs=[_row_spec(_BLK, 128), _full_spec(_K, 128), _full_spec(1, _K),
                  _row_spec(_BLK, 128), _full_spec(_K, 128), _full_spec(1, _K)],
        out_specs=[_row_spec(_BLK, _K)] * 2,
        out_shape=[out, out],
    )(fv, wv, bv, ft, wt, bt)


def _prep_users(gv, gt):
    grid = _NU // _BLK
    out = jax.ShapeDtypeStruct((_NU, _K), jnp.float32)
    return pl.pallas_call(
        _prep_users_body,
        grid=(grid,),
        in_specs=[_row_spec(_BLK, _K)] * 2,
        out_specs=[_row_spec(_BLK, _K)] * 2,
        out_shape=[out, out],
    )(gv, gt)


def _layer0(xv, pwv, gwv, lwv, bv, xt, pwt, gwt, lwt, bt, ego):
    grid = _N // _BLK
    out = jax.ShapeDtypeStruct((_N, _K), jnp.float32)
    w = _full_spec(_K, _K)
    b = _full_spec(1, _K)
    r = _row_spec(_BLK, _K)
    return pl.pallas_call(
        _layer0_body,
        grid=(grid,),
        in_specs=[r, w, w, w, b, r, w, w, w, b, r],
        out_specs=[r, r, r, r],
        out_shape=[out, out, out, out],
    )(xv, pwv, gwv, lwv, bv, xt, pwt, gwt, lwt, bt, ego)


def _layer1(sv, xpv, pwv, gwv, lwv, bv, st, xpt, pwt, gwt, lwt, bt, ego):
    grid = _N // _BLK
    out = jax.ShapeDtypeStruct((_N, _K), jnp.float32)
    w = _full_spec(_K, _K)
    b = _full_spec(1, _K)
    r = _row_spec(_BLK, _K)
    return pl.pallas_call(
        _layer1_body,
        grid=(grid,),
        in_specs=[r, r, w, w, w, b, r, r, w, w, w, b, r],
        out_specs=[r, r, r, r],
        out_shape=[out, out, out, out],
    )(sv, xpv, pwv, gwv, lwv, bv, st, xpt, pwt, gwt, lwt, bt, ego)


def _final(a, b, c, d):
    grid = _N // _BLK
    r = _row_spec(_BLK, _K)
    return pl.pallas_call(
        _final_body,
        grid=(grid,),
        in_specs=[r, r, r, r],
        out_specs=r,
        out_shape=jax.ShapeDtypeStruct((_N, _K), jnp.float32),
    )(a, b, c, d)


def kernel(edge_index, Gu, Gi, feat_visual, Gum_visual, proj_W_visual, proj_b_visual, prop_W_visual_0, lin_W_visual_0, lin_b_visual_0, g_W_visual_0, g_b_visual_0, prop_W_visual_1, lin_W_visual_1, lin_b_visual_1, g_W_visual_1, g_b_visual_1, feat_textual, Gum_textual, proj_W_textual, proj_b_textual, prop_W_textual_0, lin_W_textual_0, lin_b_textual_0, g_W_textual_0, g_b_textual_0, prop_W_textual_1, lin_W_textual_1, lin_b_textual_1, g_W_textual_1, g_b_textual_1):
    npad = _EPAD - _E
    pad_src = jnp.arange(npad, dtype=jnp.int32) & 16383
    pad_dst = jnp.full((npad,), _N, jnp.int32) + (jnp.arange(npad, dtype=jnp.int32) & 15)
    src2d = jnp.concatenate([edge_index[0], pad_src]).reshape(_EPAD // _EROW, _EROW)
    dst2d = jnp.concatenate([edge_index[1], pad_dst]).reshape(_EPAD // _EROW, _EROW)
    zz = jnp.zeros((_ACC, _K), jnp.float32)
    ego = jnp.concatenate([Gu, Gi], axis=0)

    bias0_v = (lin_b_visual_0 + g_b_visual_0).reshape(1, _K)
    bias1_v = (lin_b_visual_1 + g_b_visual_1).reshape(1, _K)
    bias0_t = (lin_b_textual_0 + g_b_textual_0).reshape(1, _K)
    bias1_t = (lin_b_textual_1 + g_b_textual_1).reshape(1, _K)

    xi_v, xi_t = _prep_items(feat_visual, proj_W_visual,
                             proj_b_visual.reshape(1, _K),
                             feat_textual, proj_W_textual,
                             proj_b_textual.reshape(1, _K))
    xu_v, xu_t = _prep_users(Gum_visual, Gum_textual)
    x0_v = jnp.concatenate([xu_v, xi_v], axis=0)
    x0_t = jnp.concatenate([xu_t, xi_t], axis=0)

    z0_v, xh0_v, z0_t, xh0_t = _layer0(
        x0_v, prop_W_visual_0, g_W_visual_0, lin_W_visual_0, bias0_v,
        x0_t, prop_W_textual_0, g_W_textual_0, lin_W_textual_0, bias0_t, ego)

    s0_v = _segment_sum_sc(z0_v, src2d, dst2d, zz)
    s0_t = _segment_sum_sc(z0_t, src2d, dst2d, zz)

    z1_v, xh1_v, z1_t, xh1_t = _layer1(
        s0_v, xh0_v, prop_W_visual_1, g_W_visual_1, lin_W_visual_1, bias1_v,
        s0_t, xh0_t, prop_W_textual_1, g_W_textual_1, lin_W_textual_1, bias1_t,
        ego)

    s1_v = _segment_sum_sc(z1_v, src2d, dst2d, zz)
    s1_t = _segment_sum_sc(z1_t, src2d, dst2d, zz)

    x_all = _final(s1_v, xh1_v, s1_t, xh1_t)
    return x_all[:_NU], x_all[_NU:]


# speedup vs baseline: 4.6325x; 1.6287x over previous
import functools

import jax
import jax.numpy as jnp
from jax import lax
from jax.experimental import pallas as pl
from jax.experimental.pallas import tpu as pltpu
from jax.experimental.pallas import tpu_sc as plsc

_NU = 25000
_NI = 25000
_N = _NU + _NI
_E = 800000
_K = 64
_BLK = 1000

_HALF = 25000
_ACC = 25088
_ROWS_PER_TILE = _ACC // 16
_EROW = 128
_GRP = 2
_EPAD = 819200
_NGRP = _EPAD // (_EROW * _GRP)
_WB = 1000


_RPT = _EPAD // _EROW // 16
_IB = 8
_NB = _RPT // _IB


def _seg_body(z, srcm, dstm, zz, out, sidx, dloc, rows, acc,
              isem0, isem1, gsem0, gsem1):
    c = lax.axis_index("c")
    s = lax.axis_index("s")
    base = c * _HALF
    isem = (isem0, isem1)
    gsem = (gsem0, gsem1)
    tile_r0 = s * _RPT

    pltpu.sync_copy(zz.at[pl.ds(s * _ROWS_PER_TILE, _ROWS_PER_TILE)],
                    acc.at[pl.ds(s * _ROWS_PER_TILE, _ROWS_PER_TILE)])
    plsc.subcore_barrier()

    iota = lax.iota(jnp.int32, 16)

    def remap(p):
        for r in range(_IB):
            for i in range(_EROW // 16):
                d = dloc[p, r, pl.ds(i * 16, 16)]
                loc = d - base
                m = (loc >= 0) & (loc < _HALF)
                dloc[p, r, pl.ds(i * 16, 16)] = jnp.where(m, loc, _HALF + iota)

    def issue_idx(p, blk):
        r0 = tile_r0 + blk * _IB
        pltpu.async_copy(srcm.at[pl.ds(r0, _IB)], sidx.at[p], isem[p])
        pltpu.async_copy(dstm.at[pl.ds(r0, _IB)], dloc.at[p], isem[p])

    def wait_idx(p):
        pltpu.make_async_copy(srcm.at[pl.ds(0, _IB)], sidx.at[p], isem[p]).wait()
        pltpu.make_async_copy(dstm.at[pl.ds(0, _IB)], dloc.at[p], isem[p]).wait()

    def issue_gather(p, r, q):
        pltpu.async_copy(z.at[sidx.at[p, r]], rows.at[q], gsem[q])

    def wait_gather(q):
        pltpu.make_async_copy(z.at[sidx.at[0, 0]], rows.at[q], gsem[q]).wait()

    pltpu.sync_copy(srcm.at[pl.ds(tile_r0, _IB)], sidx.at[0])
    pltpu.sync_copy(dstm.at[pl.ds(tile_r0, _IB)], dloc.at[0])
    remap(0)
    issue_idx(1, 1)
    issue_gather(0, 0, 0)

    def loop(i, carry):
        for p in (0, 1):
            x = 2 * i + p
            pn = p ^ 1

            @pl.when(x + 1 < _NB)
            def _():
                wait_idx(pn)
                remap(pn)
            for r in range(_IB):
                q = r & 1
                g = x * _IB + r

                @pl.when(g + 1 < _RPT)
                def _():
                    if r < _IB - 1:
                        issue_gather(p, r + 1, q ^ 1)
                    else:
                        issue_gather(pn, 0, q ^ 1)
                wait_gather(q)
                pltpu.sync_copy(rows.at[q], acc.at[dloc.at[p, r]], add=True)

            @pl.when(x + 2 < _NB)
            def _():
                issue_idx(p, x + 2)
        return carry

    lax.fori_loop(0, _NB // 2, loop, 0)
    plsc.subcore_barrier()

    for k2 in range(( _HALF // _WB + 15) // 16):
        g = s + 16 * k2

        @pl.when(g < _HALF // _WB)
        def _():
            pltpu.sync_copy(acc.at[pl.ds(g * _WB, _WB)],
                            out.at[pl.ds(base + g * _WB, _WB)])


@jax.jit
def _segment_sum_sc(z, src2d, dst2d, zz):
    mesh = plsc.VectorSubcoreMesh(core_axis_name="c", subcore_axis_name="s")
    return pl.kernel(
        _seg_body,
        out_type=jax.ShapeDtypeStruct((_N, _K), jnp.float32),
        mesh=mesh,
        scratch_types=[
            pltpu.VMEM((2, _IB, _EROW), jnp.int32),
            pltpu.VMEM((2, _IB, _EROW), jnp.int32),
            pltpu.VMEM((2, _EROW, _K), jnp.float32),
            pltpu.VMEM_SHARED((_ACC, _K), jnp.float32),
            pltpu.SemaphoreType.DMA,
            pltpu.SemaphoreType.DMA,
            pltpu.SemaphoreType.DMA,
            pltpu.SemaphoreType.DMA,
        ],
        compiler_params=pltpu.CompilerParams(use_tc_tiling_on_sc=False),
    )(z, src2d, dst2d, zz)


def _dotT(a, b):
    return lax.dot_general(a, b, (((1,), (1,)), ((), ())),
                           preferred_element_type=jnp.float32)


def _normalize(x):
    nrm = jnp.sqrt(jnp.sum(x * x, axis=1, keepdims=True))
    return x / jnp.maximum(nrm, 1e-12)


def _prep_items_body(fv, wv, bv, ft, wt, bt, ov, ot):
    ov[...] = _normalize(_dotT(fv[...], wv[...]) + bv[...])
    ot[...] = _normalize(_dotT(ft[...], wt[...]) + bt[...])


def _prep_users_body(gv, gt, ov, ot):
    ov[...] = _normalize(gv[...])
    ot[...] = _normalize(gt[...])


def _layer_core(x, pw, gw, lw, bias, ego):
    weff = _dotT(pw, gw)
    z = jnp.dot(x, weff, preferred_element_type=jnp.float32)
    xh = _dotT(x, lw) + bias + ego
    return z, xh


def _layer0_body(xv, pwv, gwv, lwv, bv, xt, pwt, gwt, lwt, bt, ego,
                 zv, xhv, zt, xht):
    zv[...], xhv[...] = _layer_core(xv[...], pwv[...], gwv[...], lwv[...],
                                    bv[...], ego[...])
    zt[...], xht[...] = _layer_core(xt[...], pwt[...], gwt[...], lwt[...],
                                    bt[...], ego[...])


def _layer1_body(sv, xpv, pwv, gwv, lwv, bv, st, xpt, pwt, gwt, lwt, bt, ego,
                 zv, xhv, zt, xht):
    zv[...], xhv[...] = _layer_core(sv[...] + xpv[...], pwv[...], gwv[...],
                                    lwv[...], bv[...], ego[...])
    zt[...], xht[...] = _layer_core(st[...] + xpt[...], pwt[...], gwt[...],
                                    lwt[...], bt[...], ego[...])


def _final_body(a, b, c, d, o):
    o[...] = a[...] + b[...] + c[...] + d[...]


def _row_spec(blk, k):
    return pl.BlockSpec((blk, k), lambda i: (i, 0))


def _full_spec(r, k):
    return pl.BlockSpec((r, k), lambda i: (0, 0))


def _prep_items(fv, wv, bv, ft, wt, bt):
    grid = _NI // _BLK
    out = jax.ShapeDtypeStruct((_NI, _K), jnp.float32)
    return pl.pallas_call(
        _prep_items_body,
        grid=(grid,),
        in_specs=[_row_spec(_BLK, 128), _full_spec(_K, 128), _full_spec(1, _K),
                  _row_spec(_BLK, 128), _full_spec(_K, 128), _full_spec(1, _K)],
        out_specs=[_row_spec(_BLK, _K)] * 2,
        out_shape=[out, out],
    )(fv, wv, bv, ft, wt, bt)


def _prep_users(gv, gt):
    grid = _NU // _BLK
    out = jax.ShapeDtypeStruct((_NU, _K), jnp.float32)
    return pl.pallas_call(
        _prep_users_body,
        grid=(grid,),
        in_specs=[_row_spec(_BLK, _K)] * 2,
        out_specs=[_row_spec(_BLK, _K)] * 2,
        out_shape=[out, out],
    )(gv, gt)


def _layer0(xv, pwv, gwv, lwv, bv, xt, pwt, gwt, lwt, bt, ego):
    grid = _N // _BLK
    out = jax.ShapeDtypeStruct((_N, _K), jnp.float32)
    w = _full_spec(_K, _K)
    b = _full_spec(1, _K)
    r = _row_spec(_BLK, _K)
    return pl.pallas_call(
        _layer0_body,
        grid=(grid,),
        in_specs=[r, w, w, w, b, r, w, w, w, b, r],
        out_specs=[r, r, r, r],
        out_shape=[out, out, out, out],
    )(xv, pwv, gwv, lwv, bv, xt, pwt, gwt, lwt, bt, ego)


def _layer1(sv, xpv, pwv, gwv, lwv, bv, st, xpt, pwt, gwt, lwt, bt, ego):
    grid = _N // _BLK
    out = jax.ShapeDtypeStruct((_N, _K), jnp.float32)
    w = _full_spec(_K, _K)
    b = _full_spec(1, _K)
    r = _row_spec(_BLK, _K)
    return pl.pallas_call(
        _layer1_body,
        grid=(grid,),
        in_specs=[r, r, w, w, w, b, r, r, w, w, w, b, r],
        out_specs=[r, r, r, r],
        out_shape=[out, out, out, out],
    )(sv, xpv, pwv, gwv, lwv, bv, st, xpt, pwt, gwt, lwt, bt, ego)


def _final(a, b, c, d):
    grid = _N // _BLK
    r = _row_spec(_BLK, _K)
    return pl.pallas_call(
        _final_body,
        grid=(grid,),
        in_specs=[r, r, r, r],
        out_specs=r,
        out_shape=jax.ShapeDtypeStruct((_N, _K), jnp.float32),
    )(a, b, c, d)


def kernel(edge_index, Gu, Gi, feat_visual, Gum_visual, proj_W_visual, proj_b_visual, prop_W_visual_0, lin_W_visual_0, lin_b_visual_0, g_W_visual_0, g_b_visual_0, prop_W_visual_1, lin_W_visual_1, lin_b_visual_1, g_W_visual_1, g_b_visual_1, feat_textual, Gum_textual, proj_W_textual, proj_b_textual, prop_W_textual_0, lin_W_textual_0, lin_b_textual_0, g_W_textual_0, g_b_textual_0, prop_W_textual_1, lin_W_textual_1, lin_b_textual_1, g_W_textual_1, g_b_textual_1):
    npad = _EPAD - _E
    pad_src = jnp.arange(npad, dtype=jnp.int32) & 16383
    pad_dst = jnp.full((npad,), _N, jnp.int32) + (jnp.arange(npad, dtype=jnp.int32) & 15)
    src2d = jnp.concatenate([edge_index[0], pad_src]).reshape(_EPAD // _EROW, _EROW)
    dst2d = jnp.concatenate([edge_index[1], pad_dst]).reshape(_EPAD // _EROW, _EROW)
    zz = jnp.zeros((_ACC, _K), jnp.float32)
    ego = jnp.concatenate([Gu, Gi], axis=0)

    bias0_v = (lin_b_visual_0 + g_b_visual_0).reshape(1, _K)
    bias1_v = (lin_b_visual_1 + g_b_visual_1).reshape(1, _K)
    bias0_t = (lin_b_textual_0 + g_b_textual_0).reshape(1, _K)
    bias1_t = (lin_b_textual_1 + g_b_textual_1).reshape(1, _K)

    xi_v, xi_t = _prep_items(feat_visual, proj_W_visual,
                             proj_b_visual.reshape(1, _K),
                             feat_textual, proj_W_textual,
                             proj_b_textual.reshape(1, _K))
    xu_v, xu_t = _prep_users(Gum_visual, Gum_textual)
    x0_v = jnp.concatenate([xu_v, xi_v], axis=0)
    x0_t = jnp.concatenate([xu_t, xi_t], axis=0)

    z0_v, xh0_v, z0_t, xh0_t = _layer0(
        x0_v, prop_W_visual_0, g_W_visual_0, lin_W_visual_0, bias0_v,
        x0_t, prop_W_textual_0, g_W_textual_0, lin_W_textual_0, bias0_t, ego)

    s0_v = _segment_sum_sc(z0_v, src2d, dst2d, zz)
    s0_t = _segment_sum_sc(z0_t, src2d, dst2d, zz)

    z1_v, xh1_v, z1_t, xh1_t = _layer1(
        s0_v, xh0_v, prop_W_visual_1, g_W_visual_1, lin_W_visual_1, bias1_v,
        s0_t, xh0_t, prop_W_textual_1, g_W_textual_1, lin_W_textual_1, bias1_t,
        ego)

    s1_v = _segment_sum_sc(z1_v, src2d, dst2d, zz)
    s1_t = _segment_sum_sc(z1_t, src2d, dst2d, zz)

    x_all = _final(s1_v, xh1_v, s1_t, xh1_t)
    return x_all[:_NU], x_all[_NU:]
